# baseline (speedup 1.0000x reference)
import functools

import jax
import jax.numpy as jnp
from jax import lax
from jax.experimental import pallas as pl
from jax.experimental.pallas import tpu as pltpu
from jax.experimental.pallas import tpu_sc as plsc

B = 4096
L = 200
DIM = 64

NC = 2
NS = 16
NW = NC * NS

N = B * L
N_W = N // NW
CHUNK = 512
N_CHUNKS = N_W // CHUNK


def _emb_body(idx_hbm, table_hbm, out_hbm, idx_v, rows_v, sem):
    wid = lax.axis_index("s") * NC + lax.axis_index("c")
    base = wid * N_W
    pltpu.sync_copy(idx_hbm.at[pl.ds(base, N_W)], idx_v)

    def body(i, _):
        pltpu.async_copy(
            table_hbm.at[idx_v.at[pl.ds(i * CHUNK, CHUNK)]],
            rows_v, sem).wait()
        pltpu.sync_copy(rows_v, out_hbm.at[pl.ds(base + i * CHUNK, CHUNK)])
        return 0

    lax.fori_loop(0, N_CHUNKS, body, 0)


@jax.jit
def _emb(idx_flat, table):
    mesh = plsc.VectorSubcoreMesh(core_axis_name="c", subcore_axis_name="s")
    fn = functools.partial(
        pl.kernel,
        mesh=mesh,
        out_type=jax.ShapeDtypeStruct((N, DIM), jnp.float32),
        scratch_types=[
            pltpu.VMEM((N_W,), jnp.int32),
            pltpu.VMEM((CHUNK, DIM), jnp.float32),
            pltpu.SemaphoreType.DMA,
        ],
        compiler_params=pltpu.CompilerParams(use_tc_tiling_on_sc=False),
    )(_emb_body)
    return fn(idx_flat, table)


def kernel(x, table):
    idx_flat = x.reshape(N).astype(jnp.int32)
    out = _emb(idx_flat, table)
    return out.reshape(B, L, DIM)


# speedup vs baseline: 1.0203x; 1.0203x over previous
import functools

import jax
import jax.numpy as jnp
from jax import lax
from jax.experimental import pallas as pl
from jax.experimental.pallas import tpu as pltpu
from jax.experimental.pallas import tpu_sc as plsc

B = 4096
L = 200
DIM = 64

NC = 2
NS = 16
NW = NC * NS

N = B * L
N_W = N // NW
CHUNK = 256
NBUF = 4
N_CHUNKS = N_W // CHUNK
N_ROUNDS = N_CHUNKS // NBUF


def _emb_body(idx_hbm, table_hbm, out_hbm, idx_v, rows_v,
              gsem, ssem):
    wid = lax.axis_index("s") * NC + lax.axis_index("c")
    base = wid * N_W
    pltpu.sync_copy(idx_hbm.at[pl.ds(base, N_W)], idx_v)

    def gather(c, b):
        pltpu.async_copy(
            table_hbm.at[idx_v.at[pl.ds(c * CHUNK, CHUNK)]],
            rows_v.at[b], gsem.at[b])

    def gather_wait(b):
        pltpu.make_async_copy(
            table_hbm.at[idx_v.at[pl.ds(0, CHUNK)]],
            rows_v.at[b], gsem.at[b]).wait()

    def store(c, b):
        pltpu.async_copy(
            rows_v.at[b], out_hbm.at[pl.ds(base + c * CHUNK, CHUNK)],
            ssem.at[b])

    def store_wait(b):
        pltpu.make_async_copy(
            rows_v.at[b], out_hbm.at[pl.ds(base, CHUNK)],
            ssem.at[b]).wait()

    for b in range(NBUF):
        gather(b, b)

    def body(r, _):
        c0 = r * NBUF
        for b in range(NBUF):
            gather_wait(b)
            store(c0 + b, b)
        for b in range(NBUF):
            store_wait(b)
            gather(c0 + NBUF + b, b)
        return 0

    lax.fori_loop(0, N_ROUNDS - 1, body, 0)

    c0 = (N_ROUNDS - 1) * NBUF
    for b in range(NBUF):
        gather_wait(b)
        store(c0 + b, b)
    for b in range(NBUF):
        store_wait(b)


@jax.jit
def _emb(idx_flat, table):
    mesh = plsc.VectorSubcoreMesh(core_axis_name="c", subcore_axis_name="s")
    fn = functools.partial(
        pl.kernel,
        mesh=mesh,
        out_type=jax.ShapeDtypeStruct((N, DIM), jnp.float32),
        scratch_types=[
            pltpu.VMEM((N_W,), jnp.int32),
            pltpu.VMEM((NBUF, CHUNK, DIM), jnp.float32),
            pltpu.SemaphoreType.DMA((NBUF,)),
            pltpu.SemaphoreType.DMA((NBUF,)),
        ],
        compiler_params=pltpu.CompilerParams(use_tc_tiling_on_sc=False),
    )(_emb_body)
    return fn(idx_flat, table)


def kernel(x, table):
    idx_flat = x.reshape(N).astype(jnp.int32)
    out = _emb(idx_flat, table)
    return out.reshape(B, L, DIM)
